# baseline (speedup 1.0000x reference)
import jax
import jax.numpy as jnp
from jax.experimental import pallas as pl
from jax.experimental.pallas import tpu as pltpu

NUM_EXPERTS = 8
TOP_K = 2
TILE = 2048


def _top2(probs):
    i1 = jnp.argmax(probs, axis=0)
    p1 = jnp.max(probs, axis=0)
    row = jax.lax.broadcasted_iota(jnp.int32, probs.shape, 0)
    masked = jnp.where(row == i1[None, :], -1.0, probs)
    i2 = jnp.argmax(masked, axis=0)
    p2 = jnp.max(masked, axis=0)
    return p1, p2, i1, i2


def _router_kernel(x0_ref, x1_ref, w_ref, p0_ref, i0_ref, p1_ref, i1_ref):
    w = w_ref[...].astype(jnp.bfloat16)
    for x_ref, p_ref, i_ref in ((x0_ref, p0_ref, i0_ref),
                                (x1_ref, p1_ref, i1_ref)):
        x = x_ref[...].astype(jnp.bfloat16)
        logits = jax.lax.dot_general(
            w, x, (((1,), (1,)), ((), ())),
            preferred_element_type=jnp.float32)

        m = jnp.max(logits, axis=0, keepdims=True)
        e = jnp.exp(logits - m)
        probs = e / jnp.sum(e, axis=0, keepdims=True)

        p1, p2, i1, i2 = _top2(probs)
        p_ref[...] = jnp.concatenate([p1[None, :], p2[None, :]], axis=0)
        i_ref[...] = jnp.concatenate([i1[None, :], i2[None, :]], axis=0).astype(jnp.int32)


def kernel(hidden_states, W):
    n, d = hidden_states.shape
    half = n // 2
    nblk = half // TILE
    p0, i0, p1, i1 = pl.pallas_call(
        _router_kernel,
        grid=(nblk,),
        in_specs=[
            pl.BlockSpec((TILE, d), lambda i: (i, 0)),
            pl.BlockSpec((TILE, d), lambda i: (i + nblk, 0)),
            pl.BlockSpec((NUM_EXPERTS, d), lambda i: (0, 0)),
        ],
        out_specs=[
            pl.BlockSpec((TOP_K, TILE), lambda i: (0, i)),
            pl.BlockSpec((TOP_K, TILE), lambda i: (0, i)),
            pl.BlockSpec((TOP_K, TILE), lambda i: (0, i)),
            pl.BlockSpec((TOP_K, TILE), lambda i: (0, i)),
        ],
        out_shape=[
            jax.ShapeDtypeStruct((TOP_K, half), jnp.float32),
            jax.ShapeDtypeStruct((TOP_K, half), jnp.int32),
            jax.ShapeDtypeStruct((TOP_K, half), jnp.float32),
            jax.ShapeDtypeStruct((TOP_K, half), jnp.int32),
        ],
        compiler_params=pltpu.CompilerParams(
            dimension_semantics=("parallel",),
        ),
    )(hidden_states, hidden_states, W)
    probs_t = jnp.concatenate([p0, p1], axis=1)
    idx_t = jnp.concatenate([i0, i1], axis=1)
    return (probs_t.T, idx_t.T)


# speedup vs baseline: 1.0949x; 1.0949x over previous
import jax
import jax.numpy as jnp
from jax.experimental import pallas as pl
from jax.experimental.pallas import tpu as pltpu

NUM_EXPERTS = 8
TOP_K = 2
TILE = 4096


def _router_kernel(x_ref, w_ref, p_ref, i_ref):
    x = x_ref[...]
    w = w_ref[...]
    logits = jax.lax.dot_general(
        w, x, (((1,), (1,)), ((), ())),
        preferred_element_type=jnp.float32)

    m = jnp.max(logits, axis=0, keepdims=True)
    e = jnp.exp(logits - m)
    probs = e / jnp.sum(e, axis=0, keepdims=True)

    i1 = jnp.argmax(probs, axis=0)
    p1 = jnp.max(probs, axis=0)
    row = jax.lax.broadcasted_iota(jnp.int32, probs.shape, 0)
    masked = jnp.where(row == i1[None, :], -1.0, probs)
    i2 = jnp.argmax(masked, axis=0)
    p2 = jnp.max(masked, axis=0)

    p_ref[...] = jnp.concatenate([p1[None, :], p2[None, :]], axis=0)
    i_ref[...] = jnp.concatenate([i1[None, :], i2[None, :]], axis=0).astype(jnp.int32)


def kernel(hidden_states, W):
    n, d = hidden_states.shape
    probs_t, idx_t = pl.pallas_call(
        _router_kernel,
        grid=(n // TILE,),
        in_specs=[
            pl.BlockSpec((TILE, d), lambda i: (i, 0)),
            pl.BlockSpec((NUM_EXPERTS, d), lambda i: (0, 0)),
        ],
        out_specs=[
            pl.BlockSpec((TOP_K, TILE), lambda i: (0, i)),
            pl.BlockSpec((TOP_K, TILE), lambda i: (0, i)),
        ],
        out_shape=[
            jax.ShapeDtypeStruct((TOP_K, n), jnp.float32),
            jax.ShapeDtypeStruct((TOP_K, n), jnp.int32),
        ],
        compiler_params=pltpu.CompilerParams(
            dimension_semantics=("parallel",),
        ),
    )(hidden_states, W)
    return (probs_t.T, idx_t.T)
